# baseline (speedup 1.0000x reference)
import functools

import jax
import jax.numpy as jnp
from jax import lax
from jax.experimental import pallas as pl
from jax.experimental.pallas import tpu as pltpu
from jax.experimental.pallas import tpu_sc as plsc

N = 10000
E = 160000
D = 256
NUM_CLASS = 8
NUM_GRAPHS = 64

NC = 2
NS = 16
NW = NC * NS
L = 16

C = 4
PASSES = D // (C * NW)
EK = 2000
NEK = E // EK
NB_E = EK // L
NB_N = N // L

E_PER_W = E // NW
N_PAD = 10240
N_PER_W = N_PAD // NW

NEG_INF = float("-inf")

_mesh = plsc.VectorSubcoreMesh(core_axis_name="c", subcore_axis_name="s")
_sc_params = pltpu.CompilerParams(needs_layout_passes=False)


def _wid():
    return lax.axis_index("s") * NC + lax.axis_index("c")


def _full(val, dtype=jnp.float32):
    return jnp.full((L,), val, dtype=dtype)


@functools.partial(
    pl.kernel,
    out_type=[
        jax.ShapeDtypeStruct((E,), jnp.float32),
        jax.ShapeDtypeStruct((N_PAD,), jnp.float32),
    ],
    mesh=_mesh,
    scratch_types=[
        pltpu.VMEM((E_PER_W,), jnp.int32),
        pltpu.VMEM((E_PER_W,), jnp.float32),
        pltpu.VMEM((N_PER_W,), jnp.int32),
        pltpu.VMEM((N_PER_W,), jnp.float32),
        pltpu.SemaphoreType.DMA,
    ],
    compiler_params=_sc_params,
)
def _gather_tables(attr_hbm, ean_hbm, nidx_hbm, etans_hbm,
                   w_hbm, eta_hbm, attr_v, w_v, nidx_v, eta_v, sem):
    wid = _wid()
    ebase = wid * E_PER_W
    pltpu.sync_copy(attr_hbm.at[pl.ds(ebase, E_PER_W)], attr_v)

    @pl.loop(0, 62)
    def _w_chunk(j):
        o = j * 80
        pltpu.async_copy(ean_hbm.at[attr_v.at[pl.ds(o, 80)]],
                         w_v.at[pl.ds(o, 80)], sem).wait()

    pltpu.async_copy(ean_hbm.at[attr_v.at[pl.ds(4960, 40)]],
                     w_v.at[pl.ds(4960, 40)], sem).wait()
    pltpu.sync_copy(w_v, w_hbm.at[pl.ds(ebase, E_PER_W)])

    nbase = wid * N_PER_W
    pltpu.sync_copy(nidx_hbm.at[pl.ds(nbase, N_PER_W)], nidx_v)

    @pl.loop(0, N_PER_W // 64)
    def _eta_chunk(j):
        o = j * 64
        pltpu.async_copy(etans_hbm.at[nidx_v.at[pl.ds(o, 64)]],
                         eta_v.at[pl.ds(o, 64)], sem).wait()

    pltpu.sync_copy(eta_v, eta_hbm.at[pl.ds(nbase, N_PER_W)])


@functools.partial(
    pl.kernel,
    out_type=jax.ShapeDtypeStruct((D * NUM_GRAPHS,), jnp.float32),
    mesh=_mesh,
    scratch_types=[
        pltpu.VMEM((C, N), jnp.float32),
        pltpu.VMEM((C, N), jnp.float32),
        pltpu.VMEM((EK,), jnp.int32),
        pltpu.VMEM((EK,), jnp.int32),
        pltpu.VMEM((EK,), jnp.float32),
        pltpu.VMEM((N,), jnp.float32),
        pltpu.VMEM((N,), jnp.int32),
        pltpu.VMEM((C * NUM_GRAPHS,), jnp.float32),
        pltpu.SemaphoreType.DMA,
    ],
    compiler_params=_sc_params,
)
def _propagate(featT_hbm, src_hbm, dst_hbm, w_hbm, eta_hbm, batch_hbm,
               gt_hbm, feat_v, r_v, src_v, dst_v, w_v, eta_v, batch_v,
               g_v, sem):
    wid = _wid()
    pltpu.sync_copy(eta_hbm.at[pl.ds(0, N)], eta_v)
    pltpu.sync_copy(batch_hbm.at[pl.ds(0, N)], batch_v)

    c_idx = [_full(c, jnp.int32) for c in range(C)]

    for p in range(PASSES):
        row0 = p * (C * NW) + wid * C
        pltpu.sync_copy(featT_hbm.at[pl.ds(row0, C)], feat_v)

        for c in range(C):
            @pl.loop(0, NB_N)
            def _init_r(i, c=c):
                r_v[c, pl.ds(i * L, L)] = _full(NEG_INF)

        @pl.loop(0, (C * NUM_GRAPHS) // L)
        def _init_g(i):
            g_v[pl.ds(i * L, L)] = _full(0.0)

        @pl.loop(0, NEK)
        def _edge_chunk(g):
            o = g * EK
            pltpu.sync_copy(src_hbm.at[pl.ds(o, EK)], src_v)
            pltpu.sync_copy(dst_hbm.at[pl.ds(o, EK)], dst_v)
            pltpu.sync_copy(w_hbm.at[pl.ds(o, EK)], w_v)

            @pl.loop(0, NB_E)
            def _edge_batch(b):
                eo = b * L
                s16 = src_v[pl.ds(eo, L)]
                d16 = dst_v[pl.ds(eo, L)]
                w16 = w_v[pl.ds(eo, L)]
                occ, _ = plsc.scan_count(d16)
                nrounds = jnp.max(occ) + 1
                vals = []
                for c in range(C):
                    f = plsc.load_gather(feat_v, [c_idx[c], s16])
                    vals.append(w16 * f)

                def _round(k, carry):
                    m = occ == k
                    for c in range(C):
                        rv = plsc.load_gather(r_v, [c_idx[c], d16])
                        plsc.store_scatter(r_v, [c_idx[c], d16],
                                           jnp.maximum(rv, vals[c]), mask=m)
                    return carry

                lax.fori_loop(0, nrounds, _round, 0)

        @pl.loop(0, NB_N)
        def _node_batch(nb):
            no = nb * L
            e16 = eta_v[pl.ds(no, L)]
            bat16 = batch_v[pl.ds(no, L)]
            occ, _ = plsc.scan_count(bat16)
            nrounds = jnp.max(occ) + 1
            xs = []
            for c in range(C):
                rv = r_v[c, pl.ds(no, L)]
                rz = jnp.where(rv == NEG_INF, 0.0, rv)
                fv = feat_v[c, pl.ds(no, L)]
                xs.append(rz + e16 * (fv - rz))

            def _round(k, carry):
                m = occ == k
                for c in range(C):
                    idx = bat16 + c * NUM_GRAPHS
                    gv = plsc.load_gather(g_v, [idx])
                    plsc.store_scatter(g_v, [idx], gv + xs[c], mask=m)
                return carry

            lax.fori_loop(0, nrounds, _round, 0)

        pltpu.sync_copy(g_v, gt_hbm.at[pl.ds(row0 * NUM_GRAPHS,
                                             C * NUM_GRAPHS)])


def _readout_body(g_ref, w_ref, b_ref, o_ref):
    g = g_ref[...]
    w = w_ref[...]
    logits = lax.dot_general(g, w, (((0,), (0,)), ((), ())),
                             preferred_element_type=jnp.float32)
    logits = logits + b_ref[...][None, :]
    m = jnp.max(logits, axis=1, keepdims=True)
    e = jnp.exp(logits - m)
    o_ref[...] = e / jnp.sum(e, axis=1, keepdims=True)


def kernel(feature, nodesindex, adj, edge_attr, batch, ean, etans, W, b):
    featT = feature.T
    src = adj[0].astype(jnp.int32)
    dst = adj[1].astype(jnp.int32)
    edge_attr = edge_attr.astype(jnp.int32)
    nidx_pad = jnp.zeros((N_PAD,), jnp.int32).at[:N].set(
        nodesindex.astype(jnp.int32))
    batch = batch.astype(jnp.int32)

    w_e, eta = _gather_tables(edge_attr, ean, nidx_pad, etans)
    gt_flat = _propagate(featT, src, dst, w_e, eta, batch)
    gT = gt_flat.reshape(D, NUM_GRAPHS)

    return pl.pallas_call(
        _readout_body,
        out_shape=jax.ShapeDtypeStruct((NUM_GRAPHS, NUM_CLASS), jnp.float32),
    )(gT, W, b)


# speedup vs baseline: 2.4406x; 2.4406x over previous
import functools

import jax
import jax.numpy as jnp
from jax import lax
from jax.experimental import pallas as pl
from jax.experimental.pallas import tpu as pltpu
from jax.experimental.pallas import tpu_sc as plsc

N = 10000
E = 160000
D = 256
NUM_CLASS = 8
NUM_GRAPHS = 64

NC = 2
NS = 16
NW = NC * NS
L = 16

P = 4
EK = 2000
NEK = E // EK
NB_E = EK // L
NB_N = N // L

E_PER_W = E // NW
N_PAD = 10240
N_PER_W = N_PAD // NW

NEG_INF = float("-inf")
NEG_INF_PAIR = -8355968

_mesh = plsc.VectorSubcoreMesh(core_axis_name="c", subcore_axis_name="s")
_sc_params = pltpu.CompilerParams(needs_layout_passes=False)
_ILV = plsc.PackFormat.INTERLEAVED


def _wid():
    return lax.axis_index("s") * NC + lax.axis_index("c")


def _full(val, dtype=jnp.float32):
    return jnp.full((L,), val, dtype=dtype)


@functools.partial(
    pl.kernel,
    out_type=[
        jax.ShapeDtypeStruct((E,), jnp.float32),
        jax.ShapeDtypeStruct((N_PAD,), jnp.float32),
    ],
    mesh=_mesh,
    scratch_types=[
        pltpu.VMEM((E_PER_W,), jnp.int32),
        pltpu.VMEM((E_PER_W,), jnp.float32),
        pltpu.VMEM((N_PER_W,), jnp.int32),
        pltpu.VMEM((N_PER_W,), jnp.float32),
        pltpu.SemaphoreType.DMA,
    ],
    compiler_params=_sc_params,
)
def _gather_tables(attr_hbm, ean_hbm, nidx_hbm, etans_hbm,
                   w_hbm, eta_hbm, attr_v, w_v, nidx_v, eta_v, sem):
    wid = _wid()
    ebase = wid * E_PER_W
    pltpu.sync_copy(attr_hbm.at[pl.ds(ebase, E_PER_W)], attr_v)

    @pl.loop(0, 62)
    def _w_chunk(j):
        o = j * 80
        pltpu.async_copy(ean_hbm.at[attr_v.at[pl.ds(o, 80)]],
                         w_v.at[pl.ds(o, 80)], sem).wait()

    pltpu.async_copy(ean_hbm.at[attr_v.at[pl.ds(4960, 40)]],
                     w_v.at[pl.ds(4960, 40)], sem).wait()
    pltpu.sync_copy(w_v, w_hbm.at[pl.ds(ebase, E_PER_W)])

    nbase = wid * N_PER_W
    pltpu.sync_copy(nidx_hbm.at[pl.ds(nbase, N_PER_W)], nidx_v)

    @pl.loop(0, N_PER_W // 64)
    def _eta_chunk(j):
        o = j * 64
        pltpu.async_copy(etans_hbm.at[nidx_v.at[pl.ds(o, 64)]],
                         eta_v.at[pl.ds(o, 64)], sem).wait()

    pltpu.sync_copy(eta_v, eta_hbm.at[pl.ds(nbase, N_PER_W)])


@functools.partial(
    pl.kernel,
    out_type=jax.ShapeDtypeStruct((D * NUM_GRAPHS,), jnp.float32),
    mesh=_mesh,
    scratch_types=[
        pltpu.VMEM((P, N), jnp.int32),
        pltpu.VMEM((P, N), jnp.int32),
        pltpu.VMEM((EK,), jnp.int32),
        pltpu.VMEM((EK,), jnp.int32),
        pltpu.VMEM((EK,), jnp.int32),
        pltpu.VMEM((EK,), jnp.int32),
        pltpu.VMEM((EK,), jnp.float32),
        pltpu.VMEM((EK,), jnp.float32),
        pltpu.VMEM((N,), jnp.float32),
        pltpu.VMEM((N,), jnp.int32),
        pltpu.VMEM((D // NW * NUM_GRAPHS,), jnp.float32),
        pltpu.SemaphoreType.DMA,
        pltpu.SemaphoreType.DMA,
    ],
    compiler_params=_sc_params,
)
def _propagate(featP_hbm, src_hbm, dst_hbm, w_hbm, eta_hbm, batch_hbm,
               gt_hbm, featP_v, rP_v, src0, src1, dst0, dst1, w0, w1,
               eta_v, batch_v, g_v, sem0, sem1):
    wid = _wid()
    pltpu.sync_copy(eta_hbm.at[pl.ds(0, N)], eta_v)
    pltpu.sync_copy(batch_hbm.at[pl.ds(0, N)], batch_v)
    pltpu.sync_copy(featP_hbm.at[pl.ds(wid * P, P)], featP_v)

    j_idx = [_full(j, jnp.int32) for j in range(P)]
    bufs = ((src0, dst0, w0, sem0), (src1, dst1, w1, sem1))

    for j in range(P):
        @pl.loop(0, NB_N)
        def _init_r(i, j=j):
            rP_v[j, pl.ds(i * L, L)] = _full(NEG_INF_PAIR, jnp.int32)

    @pl.loop(0, (D // NW * NUM_GRAPHS) // L)
    def _init_g(i):
        g_v[pl.ds(i * L, L)] = _full(0.0)

    def issue(g, par):
        sb, db, wb, sem = bufs[par]
        o = g * EK
        pltpu.async_copy(src_hbm.at[pl.ds(o, EK)], sb, sem)
        pltpu.async_copy(dst_hbm.at[pl.ds(o, EK)], db, sem)
        pltpu.async_copy(w_hbm.at[pl.ds(o, EK)], wb, sem)

    def drain(par):
        sb, db, wb, sem = bufs[par]
        pltpu.make_async_copy(src_hbm.at[pl.ds(0, EK)], sb, sem).wait()
        pltpu.make_async_copy(dst_hbm.at[pl.ds(0, EK)], db, sem).wait()
        pltpu.make_async_copy(w_hbm.at[pl.ds(0, EK)], wb, sem).wait()

    def process(par):
        sb, db, wb, _ = bufs[par]

        @pl.loop(0, NB_E, init_carry=jnp.zeros((L,), jnp.int32))
        def dirty(b, carry):
            eo = b * L
            s16 = sb[pl.ds(eo, L)]
            d16 = db[pl.ds(eo, L)]
            w16 = wb[pl.ds(eo, L)]
            occ, _ = plsc.scan_count(d16)
            wbf = plsc.pack(w16, w16, format=_ILV)
            m0 = occ == 0
            m1 = occ == 1
            for j in range(P):
                f = plsc.load_gather(featP_v, [j_idx[j], s16])
                v = wbf * plsc.bitcast(f, jnp.bfloat16)
                rv = plsc.load_gather(rP_v, [j_idx[j], d16])
                mx = jnp.maximum(plsc.bitcast(rv, jnp.bfloat16), v)
                plsc.store_scatter(rP_v, [j_idx[j], d16],
                                   plsc.bitcast(mx, jnp.int32), mask=m0)
                rv1 = plsc.load_gather(rP_v, [j_idx[j], d16])
                mx1 = jnp.maximum(plsc.bitcast(rv1, jnp.bfloat16), v)
                plsc.store_scatter(rP_v, [j_idx[j], d16],
                                   plsc.bitcast(mx1, jnp.int32), mask=m1)
            return carry | jnp.where(occ >= 2, 1, 0)

        @pl.when(jnp.max(dirty) > 0)
        def _redo():
            @pl.loop(0, NB_E)
            def _redo_batch(b):
                eo = b * L
                s16 = sb[pl.ds(eo, L)]
                d16 = db[pl.ds(eo, L)]
                w16 = wb[pl.ds(eo, L)]
                occ, _ = plsc.scan_count(d16)
                nrounds = jnp.max(occ) + 1
                wbf = plsc.pack(w16, w16, format=_ILV)
                vals = []
                for j in range(P):
                    f = plsc.load_gather(featP_v, [j_idx[j], s16])
                    vals.append(wbf * plsc.bitcast(f, jnp.bfloat16))

                def _round(k, carry):
                    m = occ == k
                    for j in range(P):
                        rv = plsc.load_gather(rP_v, [j_idx[j], d16])
                        mx = jnp.maximum(plsc.bitcast(rv, jnp.bfloat16),
                                         vals[j])
                        plsc.store_scatter(rP_v, [j_idx[j], d16],
                                           plsc.bitcast(mx, jnp.int32),
                                           mask=m)
                    return carry

                lax.fori_loop(2, nrounds, _round, 0)

    issue(0, 0)

    @pl.loop(0, NEK // 2)
    def _chunk_pair(i):
        g0 = 2 * i
        issue(g0 + 1, 1)
        drain(0)
        process(0)

        @pl.when(g0 + 2 < NEK)
        def _():
            issue(g0 + 2, 0)

        drain(1)
        process(1)

    @pl.loop(0, NB_N)
    def _node_batch(nb):
        no = nb * L
        e16 = eta_v[pl.ds(no, L)]
        bat16 = batch_v[pl.ds(no, L)]
        for j in range(P):
            rp = rP_v[j, pl.ds(no, L)]
            ra, rb = plsc.unpack(plsc.bitcast(rp, jnp.bfloat16), format=_ILV)
            fp = featP_v[j, pl.ds(no, L)]
            fa, fb = plsc.unpack(plsc.bitcast(fp, jnp.bfloat16), format=_ILV)
            for col, (rcol, fcol) in enumerate(((ra, fa), (rb, fb))):
                rz = jnp.where(rcol == NEG_INF, 0.0, rcol)
                x = rz + e16 * (fcol - rz)
                idx = bat16 + (2 * j + col) * NUM_GRAPHS
                plsc.addupdate_scatter(g_v, [idx], x)

    pltpu.sync_copy(g_v, gt_hbm.at[pl.ds(wid * (D // NW * NUM_GRAPHS),
                                         D // NW * NUM_GRAPHS)])


def _readout_body(g_ref, w_ref, b_ref, o_ref):
    g = g_ref[...]
    w = w_ref[...]
    logits = lax.dot_general(g, w, (((0,), (0,)), ((), ())),
                             preferred_element_type=jnp.float32)
    logits = logits + b_ref[...][None, :]
    m = jnp.max(logits, axis=1, keepdims=True)
    e = jnp.exp(logits - m)
    o_ref[...] = e / jnp.sum(e, axis=1, keepdims=True)


def kernel(feature, nodesindex, adj, edge_attr, batch, ean, etans, W, b):
    fpair = feature.astype(jnp.bfloat16).reshape(N, D // 2, 2)
    featP = lax.bitcast_convert_type(fpair, jnp.int32).T
    src = adj[0].astype(jnp.int32)
    dst = adj[1].astype(jnp.int32)
    edge_attr = edge_attr.astype(jnp.int32)
    nidx_pad = jnp.zeros((N_PAD,), jnp.int32).at[:N].set(
        nodesindex.astype(jnp.int32))
    batch = batch.astype(jnp.int32)

    w_e, eta = _gather_tables(edge_attr, ean, nidx_pad, etans)
    gt_flat = _propagate(featP, src, dst, w_e, eta, batch)
    gT = gt_flat.reshape(D, NUM_GRAPHS)

    return pl.pallas_call(
        _readout_body,
        out_shape=jax.ShapeDtypeStruct((NUM_GRAPHS, NUM_CLASS), jnp.float32),
    )(gT, W, b)


# speedup vs baseline: 3.1913x; 1.3076x over previous
import functools

import jax
import jax.numpy as jnp
from jax import lax
from jax.experimental import pallas as pl
from jax.experimental.pallas import tpu as pltpu
from jax.experimental.pallas import tpu_sc as plsc

N = 10000
E = 160000
D = 256
NUM_CLASS = 8
NUM_GRAPHS = 64

NC = 2
NS = 16
NW = NC * NS
L = 16

P = 4
EK = 2000
NEK = E // EK
NB_E = EK // L
NB_N = N // L

E_PER_W = E // NW
N_PAD = 10240
N_PER_W = N_PAD // NW

NEG_INF = float("-inf")
NEG_INF_PAIR = -8355968

_mesh = plsc.VectorSubcoreMesh(core_axis_name="c", subcore_axis_name="s")
_sc_params = pltpu.CompilerParams(needs_layout_passes=False)
_ILV = plsc.PackFormat.INTERLEAVED


def _wid():
    return lax.axis_index("s") * NC + lax.axis_index("c")


def _full(val, dtype=jnp.float32):
    return jnp.full((L,), val, dtype=dtype)


@functools.partial(
    pl.kernel,
    out_type=[
        jax.ShapeDtypeStruct((E,), jnp.float32),
        jax.ShapeDtypeStruct((N_PAD,), jnp.float32),
    ],
    mesh=_mesh,
    scratch_types=[
        pltpu.VMEM((E_PER_W,), jnp.int32),
        pltpu.VMEM((E_PER_W,), jnp.float32),
        pltpu.VMEM((N_PER_W,), jnp.int32),
        pltpu.VMEM((N_PER_W,), jnp.float32),
        pltpu.SemaphoreType.DMA,
    ],
    compiler_params=_sc_params,
)
def _gather_tables(attr_hbm, ean_hbm, nidx_hbm, etans_hbm,
                   w_hbm, eta_hbm, attr_v, w_v, nidx_v, eta_v, sem):
    wid = _wid()
    ebase = wid * E_PER_W
    pltpu.sync_copy(attr_hbm.at[pl.ds(ebase, E_PER_W)], attr_v)

    @pl.loop(0, 62)
    def _w_chunk(j):
        o = j * 80
        pltpu.async_copy(ean_hbm.at[attr_v.at[pl.ds(o, 80)]],
                         w_v.at[pl.ds(o, 80)], sem).wait()

    pltpu.async_copy(ean_hbm.at[attr_v.at[pl.ds(4960, 40)]],
                     w_v.at[pl.ds(4960, 40)], sem).wait()
    pltpu.sync_copy(w_v, w_hbm.at[pl.ds(ebase, E_PER_W)])

    nbase = wid * N_PER_W
    pltpu.sync_copy(nidx_hbm.at[pl.ds(nbase, N_PER_W)], nidx_v)

    @pl.loop(0, N_PER_W // 64)
    def _eta_chunk(j):
        o = j * 64
        pltpu.async_copy(etans_hbm.at[nidx_v.at[pl.ds(o, 64)]],
                         eta_v.at[pl.ds(o, 64)], sem).wait()

    pltpu.sync_copy(eta_v, eta_hbm.at[pl.ds(nbase, N_PER_W)])


@functools.partial(
    pl.kernel,
    out_type=jax.ShapeDtypeStruct((D * NUM_GRAPHS,), jnp.float32),
    mesh=_mesh,
    scratch_types=[
        pltpu.VMEM((N,), jnp.int32),
        pltpu.VMEM((N,), jnp.int32),
        pltpu.VMEM((N,), jnp.int32),
        pltpu.VMEM((N,), jnp.int32),
        pltpu.VMEM((N,), jnp.int32),
        pltpu.VMEM((N,), jnp.int32),
        pltpu.VMEM((N,), jnp.int32),
        pltpu.VMEM((N,), jnp.int32),
        pltpu.VMEM((EK,), jnp.int32),
        pltpu.VMEM((EK,), jnp.int32),
        pltpu.VMEM((EK,), jnp.int32),
        pltpu.VMEM((EK,), jnp.int32),
        pltpu.VMEM((EK,), jnp.float32),
        pltpu.VMEM((EK,), jnp.float32),
        pltpu.VMEM((N,), jnp.float32),
        pltpu.VMEM((N,), jnp.int32),
        pltpu.VMEM((D // NW * NUM_GRAPHS,), jnp.float32),
        pltpu.SemaphoreType.DMA,
        pltpu.SemaphoreType.DMA,
    ],
    compiler_params=_sc_params,
)
def _propagate(featP_hbm, src_hbm, dst_hbm, w_hbm, eta_hbm, batch_hbm,
               gt_hbm, fp0, fp1, fp2, fp3, rp0, rp1, rp2, rp3,
               src0, src1, dst0, dst1, w0, w1,
               eta_v, batch_v, g_v, sem0, sem1):
    wid = _wid()
    feat = (fp0, fp1, fp2, fp3)
    rmax = (rp0, rp1, rp2, rp3)
    pltpu.sync_copy(eta_hbm.at[pl.ds(0, N)], eta_v)
    pltpu.sync_copy(batch_hbm.at[pl.ds(0, N)], batch_v)
    for j in range(P):
        pltpu.sync_copy(featP_hbm.at[wid * P + j], feat[j])

    bufs = ((src0, dst0, w0, sem0), (src1, dst1, w1, sem1))

    for j in range(P):
        @pl.loop(0, NB_N)
        def _init_r(i, j=j):
            rmax[j][pl.ds(i * L, L)] = _full(NEG_INF_PAIR, jnp.int32)

    @pl.loop(0, (D // NW * NUM_GRAPHS) // L)
    def _init_g(i):
        g_v[pl.ds(i * L, L)] = _full(0.0)

    def issue(g, par):
        sb, db, wb, sem = bufs[par]
        o = g * EK
        pltpu.async_copy(src_hbm.at[pl.ds(o, EK)], sb, sem)
        pltpu.async_copy(dst_hbm.at[pl.ds(o, EK)], db, sem)
        pltpu.async_copy(w_hbm.at[pl.ds(o, EK)], wb, sem)

    def drain(par):
        sb, db, wb, sem = bufs[par]
        pltpu.make_async_copy(src_hbm.at[pl.ds(0, EK)], sb, sem).wait()
        pltpu.make_async_copy(dst_hbm.at[pl.ds(0, EK)], db, sem).wait()
        pltpu.make_async_copy(w_hbm.at[pl.ds(0, EK)], wb, sem).wait()

    def process(par):
        sb, db, wb, _ = bufs[par]

        @pl.loop(0, NB_E, init_carry=jnp.zeros((L,), jnp.int32))
        def dirty(b, carry):
            eo = b * L
            s16 = sb[pl.ds(eo, L)]
            d16 = db[pl.ds(eo, L)]
            w16 = wb[pl.ds(eo, L)]
            occ, _ = plsc.scan_count(d16)
            wbf = plsc.pack(w16, w16, format=_ILV)
            m0 = occ == 0
            m1 = occ == 1
            fs = [plsc.load_gather(feat[j], [s16]) for j in range(P)]
            rvs = [plsc.load_gather(rmax[j], [d16]) for j in range(P)]
            vs = [wbf * plsc.bitcast(fs[j], jnp.bfloat16) for j in range(P)]
            mxs = [plsc.bitcast(
                jnp.maximum(plsc.bitcast(rvs[j], jnp.bfloat16), vs[j]),
                jnp.int32) for j in range(P)]
            for j in range(P):
                plsc.store_scatter(rmax[j], [d16], mxs[j], mask=m0)
            rv1s = [plsc.load_gather(rmax[j], [d16]) for j in range(P)]
            mx1s = [plsc.bitcast(
                jnp.maximum(plsc.bitcast(rv1s[j], jnp.bfloat16), vs[j]),
                jnp.int32) for j in range(P)]
            for j in range(P):
                plsc.store_scatter(rmax[j], [d16], mx1s[j], mask=m1)
            return jnp.maximum(carry, occ)

        @pl.when(jnp.max(dirty) >= 2)
        def _redo():
            @pl.loop(0, NB_E)
            def _redo_batch(b):
                eo = b * L
                s16 = sb[pl.ds(eo, L)]
                d16 = db[pl.ds(eo, L)]
                w16 = wb[pl.ds(eo, L)]
                occ, _ = plsc.scan_count(d16)
                nrounds = jnp.max(occ) + 1
                wbf = plsc.pack(w16, w16, format=_ILV)
                vals = []
                for j in range(P):
                    f = plsc.load_gather(feat[j], [s16])
                    vals.append(wbf * plsc.bitcast(f, jnp.bfloat16))

                def _round(k, carry):
                    m = occ == k
                    for j in range(P):
                        rv = plsc.load_gather(rmax[j], [d16])
                        mx = jnp.maximum(plsc.bitcast(rv, jnp.bfloat16),
                                         vals[j])
                        plsc.store_scatter(rmax[j], [d16],
                                           plsc.bitcast(mx, jnp.int32),
                                           mask=m)
                    return carry

                lax.fori_loop(2, nrounds, _round, 0)

    issue(0, 0)

    @pl.loop(0, NEK // 2)
    def _chunk_pair(i):
        g0 = 2 * i
        issue(g0 + 1, 1)
        drain(0)
        process(0)

        @pl.when(g0 + 2 < NEK)
        def _():
            issue(g0 + 2, 0)

        drain(1)
        process(1)

    @pl.loop(0, NB_N)
    def _node_batch(nb):
        no = nb * L
        e16 = eta_v[pl.ds(no, L)]
        bat16 = batch_v[pl.ds(no, L)]
        for j in range(P):
            rp = rmax[j][pl.ds(no, L)]
            ra, rb = plsc.unpack(plsc.bitcast(rp, jnp.bfloat16), format=_ILV)
            fp = feat[j][pl.ds(no, L)]
            fa, fb = plsc.unpack(plsc.bitcast(fp, jnp.bfloat16), format=_ILV)
            for col, (rcol, fcol) in enumerate(((ra, fa), (rb, fb))):
                rz = jnp.where(rcol == NEG_INF, 0.0, rcol)
                x = rz + e16 * (fcol - rz)
                idx = bat16 + (2 * j + col) * NUM_GRAPHS
                plsc.addupdate_scatter(g_v, [idx], x)

    pltpu.sync_copy(g_v, gt_hbm.at[pl.ds(wid * (D // NW * NUM_GRAPHS),
                                         D // NW * NUM_GRAPHS)])


def _readout_body(g_ref, w_ref, b_ref, o_ref):
    g = g_ref[...]
    w = w_ref[...]
    logits = lax.dot_general(g, w, (((0,), (0,)), ((), ())),
                             preferred_element_type=jnp.float32)
    logits = logits + b_ref[...][None, :]
    m = jnp.max(logits, axis=1, keepdims=True)
    e = jnp.exp(logits - m)
    o_ref[...] = e / jnp.sum(e, axis=1, keepdims=True)


def kernel(feature, nodesindex, adj, edge_attr, batch, ean, etans, W, b):
    fpair = feature.astype(jnp.bfloat16).reshape(N, D // 2, 2)
    featP = lax.bitcast_convert_type(fpair, jnp.int32).T
    src = adj[0].astype(jnp.int32)
    dst = adj[1].astype(jnp.int32)
    edge_attr = edge_attr.astype(jnp.int32)
    nidx_pad = jnp.zeros((N_PAD,), jnp.int32).at[:N].set(
        nodesindex.astype(jnp.int32))
    batch = batch.astype(jnp.int32)

    w_e, eta = _gather_tables(edge_attr, ean, nidx_pad, etans)
    gt_flat = _propagate(featP, src, dst, w_e, eta, batch)
    gT = gt_flat.reshape(D, NUM_GRAPHS)

    return pl.pallas_call(
        _readout_body,
        out_shape=jax.ShapeDtypeStruct((NUM_GRAPHS, NUM_CLASS), jnp.float32),
    )(gT, W, b)


# speedup vs baseline: 3.2034x; 1.0038x over previous
import functools

import jax
import jax.numpy as jnp
from jax import lax
from jax.experimental import pallas as pl
from jax.experimental.pallas import tpu as pltpu
from jax.experimental.pallas import tpu_sc as plsc

N = 10000
E = 160000
D = 256
NUM_CLASS = 8
NUM_GRAPHS = 64

NC = 2
NS = 16
NW = NC * NS
L = 16

P = 4
EK = 2000
NEK = E // EK
NB_E = EK // L
NB_N = N // L

E_PER_W = E // NW
N_PAD = 10240
N_PER_W = N_PAD // NW

NEG_INF = float("-inf")
NEG_INF_PAIR = -8355968

_mesh = plsc.VectorSubcoreMesh(core_axis_name="c", subcore_axis_name="s")
_sc_params = pltpu.CompilerParams(needs_layout_passes=False)
_ILV = plsc.PackFormat.INTERLEAVED


def _wid():
    return lax.axis_index("s") * NC + lax.axis_index("c")


def _full(val, dtype=jnp.float32):
    return jnp.full((L,), val, dtype=dtype)


@functools.partial(
    pl.kernel,
    out_type=[
        jax.ShapeDtypeStruct((E,), jnp.float32),
        jax.ShapeDtypeStruct((N_PAD,), jnp.float32),
    ],
    mesh=_mesh,
    scratch_types=[
        pltpu.VMEM((E_PER_W,), jnp.int32),
        pltpu.VMEM((E_PER_W,), jnp.float32),
        pltpu.VMEM((N_PER_W,), jnp.int32),
        pltpu.VMEM((N_PER_W,), jnp.float32),
        pltpu.SemaphoreType.DMA,
    ],
    compiler_params=_sc_params,
)
def _gather_tables(attr_hbm, ean_hbm, nidx_hbm, etans_hbm,
                   w_hbm, eta_hbm, attr_v, w_v, nidx_v, eta_v, sem):
    wid = _wid()
    ebase = wid * E_PER_W
    pltpu.sync_copy(attr_hbm.at[pl.ds(ebase, E_PER_W)], attr_v)

    @pl.loop(0, 62)
    def _w_chunk(j):
        o = j * 80
        pltpu.async_copy(ean_hbm.at[attr_v.at[pl.ds(o, 80)]],
                         w_v.at[pl.ds(o, 80)], sem).wait()

    pltpu.async_copy(ean_hbm.at[attr_v.at[pl.ds(4960, 40)]],
                     w_v.at[pl.ds(4960, 40)], sem).wait()
    pltpu.sync_copy(w_v, w_hbm.at[pl.ds(ebase, E_PER_W)])

    nbase = wid * N_PER_W
    pltpu.sync_copy(nidx_hbm.at[pl.ds(nbase, N_PER_W)], nidx_v)

    @pl.loop(0, N_PER_W // 64)
    def _eta_chunk(j):
        o = j * 64
        pltpu.async_copy(etans_hbm.at[nidx_v.at[pl.ds(o, 64)]],
                         eta_v.at[pl.ds(o, 64)], sem).wait()

    pltpu.sync_copy(eta_v, eta_hbm.at[pl.ds(nbase, N_PER_W)])


@functools.partial(
    pl.kernel,
    out_type=jax.ShapeDtypeStruct((D * NUM_GRAPHS,), jnp.float32),
    mesh=_mesh,
    scratch_types=[
        pltpu.VMEM((N,), jnp.int32),
        pltpu.VMEM((N,), jnp.int32),
        pltpu.VMEM((N,), jnp.int32),
        pltpu.VMEM((N,), jnp.int32),
        pltpu.VMEM((N,), jnp.int32),
        pltpu.VMEM((N,), jnp.int32),
        pltpu.VMEM((N,), jnp.int32),
        pltpu.VMEM((N,), jnp.int32),
        pltpu.VMEM((EK,), jnp.int32),
        pltpu.VMEM((EK,), jnp.int32),
        pltpu.VMEM((EK,), jnp.int32),
        pltpu.VMEM((EK,), jnp.int32),
        pltpu.VMEM((EK,), jnp.float32),
        pltpu.VMEM((EK,), jnp.float32),
        pltpu.VMEM((N,), jnp.float32),
        pltpu.VMEM((N,), jnp.int32),
        pltpu.VMEM((D // NW * NUM_GRAPHS,), jnp.float32),
        pltpu.SemaphoreType.DMA,
        pltpu.SemaphoreType.DMA,
    ],
    compiler_params=_sc_params,
)
def _propagate(featP_hbm, src_hbm, dst_hbm, w_hbm, eta_hbm, batch_hbm,
               gt_hbm, fp0, fp1, fp2, fp3, rp0, rp1, rp2, rp3,
               src0, src1, dst0, dst1, w0, w1,
               eta_v, batch_v, g_v, sem0, sem1):
    wid = _wid()
    feat = (fp0, fp1, fp2, fp3)
    rmax = (rp0, rp1, rp2, rp3)
    bufs = ((src0, dst0, w0, sem0), (src1, dst1, w1, sem1))

    pltpu.sync_copy(eta_hbm.at[pl.ds(0, N)], eta_v)
    pltpu.sync_copy(batch_hbm.at[pl.ds(0, N)], batch_v)
    for j in range(P):
        pltpu.sync_copy(featP_hbm.at[wid * P + j], feat[j])

    for j in range(P):
        @pl.loop(0, NB_N)
        def _init_r(i, j=j):
            rmax[j][pl.ds(i * L, L)] = _full(NEG_INF_PAIR, jnp.int32)

    @pl.loop(0, (D // NW * NUM_GRAPHS) // L)
    def _init_g(i):
        g_v[pl.ds(i * L, L)] = _full(0.0)

    def issue(g, par):
        sb, db, wb, sem = bufs[par]
        o = g * EK
        pltpu.async_copy(src_hbm.at[pl.ds(o, EK)], sb, sem)
        pltpu.async_copy(dst_hbm.at[pl.ds(o, EK)], db, sem)
        pltpu.async_copy(w_hbm.at[pl.ds(o, EK)], wb, sem)

    def drain(par):
        sb, db, wb, sem = bufs[par]
        pltpu.make_async_copy(src_hbm.at[pl.ds(0, EK)], sb, sem).wait()
        pltpu.make_async_copy(dst_hbm.at[pl.ds(0, EK)], db, sem).wait()
        pltpu.make_async_copy(w_hbm.at[pl.ds(0, EK)], wb, sem).wait()

    def process(par):
        sb, db, wb, _ = bufs[par]

        def load_front(b):
            eo = b * L
            s16 = sb[pl.ds(eo, L)]
            d16 = db[pl.ds(eo, L)]
            w16 = wb[pl.ds(eo, L)]
            fs = tuple(plsc.load_gather(feat[j], [s16]) for j in range(P))
            return (d16, w16) + fs

        def apply_batch(fr):
            d16, w16 = fr[0], fr[1]
            fs = fr[2:]
            occ, _ = plsc.scan_count(d16)
            wbf = plsc.pack(w16, w16, format=_ILV)
            vs = [wbf * plsc.bitcast(fs[j], jnp.bfloat16)
                  for j in range(P)]
            m0 = occ == 0
            m1 = occ == 1
            rvs = [plsc.load_gather(rmax[j], [d16]) for j in range(P)]
            mxs = [plsc.bitcast(
                jnp.maximum(plsc.bitcast(rvs[j], jnp.bfloat16), vs[j]),
                jnp.int32) for j in range(P)]
            for j in range(P):
                plsc.store_scatter(rmax[j], [d16], mxs[j], mask=m0)
            rv1s = [plsc.load_gather(rmax[j], [d16]) for j in range(P)]
            mx1s = [plsc.bitcast(
                jnp.maximum(plsc.bitcast(rv1s[j], jnp.bfloat16), vs[j]),
                jnp.int32) for j in range(P)]
            for j in range(P):
                plsc.store_scatter(rmax[j], [d16], mx1s[j], mask=m1)
            return occ

        init = (jnp.zeros((L,), jnp.int32),) + load_front(0)

        @pl.loop(0, NB_E - 1, init_carry=init)
        def carry_out(b, carry):
            occ = apply_batch(carry[1:])
            nxt = load_front(b + 1)
            return (jnp.maximum(carry[0], occ),) + nxt

        dirty = jnp.maximum(carry_out[0], apply_batch(carry_out[1:]))

        @pl.when(jnp.max(dirty) >= 2)
        def _redo():
            @pl.loop(0, NB_E)
            def _redo_batch(b):
                eo = b * L
                s16 = sb[pl.ds(eo, L)]
                d16 = db[pl.ds(eo, L)]
                w16 = wb[pl.ds(eo, L)]
                occ, _ = plsc.scan_count(d16)
                nrounds = jnp.max(occ) + 1
                wbf = plsc.pack(w16, w16, format=_ILV)
                vals = []
                for j in range(P):
                    f = plsc.load_gather(feat[j], [s16])
                    vals.append(wbf * plsc.bitcast(f, jnp.bfloat16))

                def _round(k, carry):
                    m = occ == k
                    for j in range(P):
                        rv = plsc.load_gather(rmax[j], [d16])
                        mx = jnp.maximum(plsc.bitcast(rv, jnp.bfloat16),
                                         vals[j])
                        plsc.store_scatter(rmax[j], [d16],
                                           plsc.bitcast(mx, jnp.int32),
                                           mask=m)
                    return carry

                lax.fori_loop(2, nrounds, _round, 0)

    issue(0, 0)

    @pl.loop(0, NEK // 2)
    def _chunk_pair(i):
        g0 = 2 * i
        issue(g0 + 1, 1)
        drain(0)
        process(0)

        @pl.when(g0 + 2 < NEK)
        def _():
            issue(g0 + 2, 0)

        drain(1)
        process(1)

    @pl.loop(0, NB_N)
    def _node_batch(nb):
        no = nb * L
        e16 = eta_v[pl.ds(no, L)]
        bat16 = batch_v[pl.ds(no, L)]
        for j in range(P):
            rp = rmax[j][pl.ds(no, L)]
            ra, rb = plsc.unpack(plsc.bitcast(rp, jnp.bfloat16), format=_ILV)
            fp = feat[j][pl.ds(no, L)]
            fa, fb = plsc.unpack(plsc.bitcast(fp, jnp.bfloat16), format=_ILV)
            for col, (rcol, fcol) in enumerate(((ra, fa), (rb, fb))):
                rz = jnp.where(rcol == NEG_INF, 0.0, rcol)
                x = rz + e16 * (fcol - rz)
                idx = bat16 + (2 * j + col) * NUM_GRAPHS
                plsc.addupdate_scatter(g_v, [idx], x)

    pltpu.sync_copy(g_v, gt_hbm.at[pl.ds(wid * (D // NW * NUM_GRAPHS),
                                         D // NW * NUM_GRAPHS)])


def _readout_body(g_ref, w_ref, b_ref, o_ref):
    g = g_ref[...]
    w = w_ref[...]
    logits = lax.dot_general(g, w, (((0,), (0,)), ((), ())),
                             preferred_element_type=jnp.float32)
    logits = logits + b_ref[...][None, :]
    m = jnp.max(logits, axis=1, keepdims=True)
    e = jnp.exp(logits - m)
    o_ref[...] = e / jnp.sum(e, axis=1, keepdims=True)


def kernel(feature, nodesindex, adj, edge_attr, batch, ean, etans, W, b):
    fpair = feature.astype(jnp.bfloat16).reshape(N, D // 2, 2)
    featP = lax.bitcast_convert_type(fpair, jnp.int32).T
    src = adj[0].astype(jnp.int32)
    dst = adj[1].astype(jnp.int32)
    edge_attr = edge_attr.astype(jnp.int32)
    nidx_pad = jnp.zeros((N_PAD,), jnp.int32).at[:N].set(
        nodesindex.astype(jnp.int32))
    batch = batch.astype(jnp.int32)

    w_e, eta = _gather_tables(edge_attr, ean, nidx_pad, etans)
    gt_flat = _propagate(featP, src, dst, w_e, eta, batch)
    gT = gt_flat.reshape(D, NUM_GRAPHS)

    return pl.pallas_call(
        _readout_body,
        out_shape=jax.ShapeDtypeStruct((NUM_GRAPHS, NUM_CLASS), jnp.float32),
    )(gT, W, b)
